# Initial kernel scaffold; baseline (speedup 1.0000x reference)
import jax
import jax.numpy as jnp
from jax.experimental import pallas as pl


def kernel(x, edge_index, batch, W0, b0, W1, b1, W2, b2, Wg, bg, Wp, bp):
    raise NotImplementedError("write your pallas kernel here")


# speedup vs baseline: 3.9953x; 3.9953x over previous
import functools

import jax
import jax.numpy as jnp
from jax import lax
from jax.experimental import pallas as pl
from jax.experimental.pallas import tpu as pltpu
from jax.experimental.pallas import tpu_sc as plsc

N = 50000
E = 800000
B = 1000
IN_F = 82
H = 128
D = 256

NC = 2
NS = 16
NW = NC * NS
FC = 32
NCHUNK = H // FC
K = 128
N_PAD = 50048
ROWS_SUB = N_PAD // NS
E_PAD = NW * 196 * K
STEPS = E_PAD // (NW * K)
B_PAD = 1024
RB = N_PAD // 128


def _make_sc_scatter(nchunks, gather):
    mesh = plsc.VectorSubcoreMesh(core_axis_name="c", subcore_axis_name="s")

    @functools.partial(
        pl.kernel,
        mesh=mesh,
        out_type=jax.ShapeDtypeStruct((NC, nchunks, N_PAD, FC), jnp.float32),
        scratch_types=[
            pltpu.VMEM((K,), jnp.int32),
            pltpu.VMEM((K,), jnp.int32),
            pltpu.VMEM((K, FC), jnp.float32),
            pltpu.VMEM_SHARED((N_PAD, FC), jnp.float32),
            pltpu.SemaphoreType.DMA,
        ],
        compiler_params=pltpu.CompilerParams(use_tc_tiling_on_sc=False),
    )
    def sc_kernel(src_hbm, dst_hbm, zeros_hbm, ones_hbm, *rest):
        g_refs = rest[:nchunks]
        out_hbm = rest[nchunks]
        srcv, dstv, rows, acc, sem = rest[nchunks + 1:]
        cid = lax.axis_index("c")
        sid = lax.axis_index("s")
        wid = sid * NC + cid
        ebase = wid * (STEPS * K)
        if not gather:
            pltpu.sync_copy(ones_hbm, rows)
        for fc in range(nchunks):
            pltpu.sync_copy(
                zeros_hbm, acc.at[pl.ds(sid * ROWS_SUB, ROWS_SUB)])
            plsc.subcore_barrier()

            def body(i, _, fc=fc):
                off = ebase + i * K
                pltpu.sync_copy(dst_hbm.at[pl.ds(off, K)], dstv)
                if gather:
                    pltpu.sync_copy(src_hbm.at[pl.ds(off, K)], srcv)
                    pltpu.async_copy(g_refs[fc].at[srcv], rows, sem).wait()
                pltpu.sync_copy(rows, acc.at[dstv], add=True)
                return 0

            lax.fori_loop(0, STEPS, body, 0)
            plsc.subcore_barrier()
            pltpu.sync_copy(
                acc.at[pl.ds(sid * ROWS_SUB, ROWS_SUB)],
                out_hbm.at[cid, fc, pl.ds(sid * ROWS_SUB, ROWS_SUB)])
            plsc.subcore_barrier()

    return sc_kernel


_sc_spmm = _make_sc_scatter(NCHUNK, gather=True)
_sc_deg = _make_sc_scatter(1, gather=False)


def _k_pre(x_ref, w_ref, degt_ref, o_ref):
    dinv = jax.lax.rsqrt(degt_ref[:, 0:1] + 1.0)
    o_ref[...] = dinv * jnp.dot(
        x_ref[...], w_ref[...], preferred_element_type=jnp.float32)


def _pre(xp, w, degt):
    return pl.pallas_call(
        _k_pre,
        grid=(RB,),
        in_specs=[
            pl.BlockSpec((128, 128), lambda i: (i, 0)),
            pl.BlockSpec((128, 128), lambda i: (0, 0)),
            pl.BlockSpec((128, 128), lambda i: (i, 0)),
        ],
        out_specs=pl.BlockSpec((128, 128), lambda i: (i, 0)),
        out_shape=jax.ShapeDtypeStruct((N_PAD, 128), jnp.float32),
    )(xp, w, degt)


def _k_mid(y_ref, g_ref, degt_ref, b_ref, w_ref, o_ref):
    dinv = jax.lax.rsqrt(degt_ref[:, 0:1] + 1.0)
    h = jnp.maximum(dinv * (y_ref[...] + g_ref[...]) + b_ref[0:1, :], 0.0)
    o_ref[...] = dinv * jnp.dot(
        h, w_ref[...], preferred_element_type=jnp.float32)


def _mid(y, g, degt, b, w):
    b8 = jnp.broadcast_to(b.reshape(1, 128), (8, 128))
    return pl.pallas_call(
        _k_mid,
        grid=(RB,),
        in_specs=[
            pl.BlockSpec((128, 128), lambda i: (i, 0)),
            pl.BlockSpec((128, 128), lambda i: (i, 0)),
            pl.BlockSpec((128, 128), lambda i: (i, 0)),
            pl.BlockSpec((8, 128), lambda i: (0, 0)),
            pl.BlockSpec((128, 128), lambda i: (0, 0)),
        ],
        out_specs=pl.BlockSpec((128, 128), lambda i: (i, 0)),
        out_shape=jax.ShapeDtypeStruct((N_PAD, 128), jnp.float32),
    )(y, g, degt, b8, w)


def _k_post(y_ref, g_ref, degt_ref, b_ref, wg_ref, h_ref, gate_ref):
    dinv = jax.lax.rsqrt(degt_ref[:, 0:1] + 1.0)
    h = jnp.maximum(dinv * (y_ref[...] + g_ref[...]) + b_ref[0:1, :], 0.0)
    h_ref[...] = h
    gate = jnp.dot(h, wg_ref[...], preferred_element_type=jnp.float32)
    gate_ref[...] = jnp.broadcast_to(gate, (128, 128))


def _post(y, g, degt, b, wg):
    b8 = jnp.broadcast_to(b.reshape(1, 128), (8, 128))
    return pl.pallas_call(
        _k_post,
        grid=(RB,),
        in_specs=[
            pl.BlockSpec((128, 128), lambda i: (i, 0)),
            pl.BlockSpec((128, 128), lambda i: (i, 0)),
            pl.BlockSpec((128, 128), lambda i: (i, 0)),
            pl.BlockSpec((8, 128), lambda i: (0, 0)),
            pl.BlockSpec((128, 1), lambda i: (0, 0)),
        ],
        out_specs=[
            pl.BlockSpec((128, 128), lambda i: (i, 0)),
            pl.BlockSpec((128, 128), lambda i: (i, 0)),
        ],
        out_shape=[
            jax.ShapeDtypeStruct((N_PAD, 128), jnp.float32),
            jax.ShapeDtypeStruct((N_PAD, 128), jnp.float32),
        ],
    )(y, g, degt, b8, wg)


def _k_segmax(gate_ref, batcht_ref, o_ref):
    i = pl.program_id(0)
    bt = batcht_ref[:, 0:1]
    lane = lax.broadcasted_iota(jnp.int32, (128, B_PAD), 1)
    oh = bt == lane
    gcol = gate_ref[:, 0:1]
    masked = jnp.where(oh, jnp.broadcast_to(gcol, (128, B_PAD)), -3e38)
    m = jnp.max(masked, axis=0, keepdims=True)
    m8 = jnp.broadcast_to(m, (8, B_PAD))

    @pl.when(i == 0)
    def _():
        o_ref[...] = m8

    @pl.when(i > 0)
    def _():
        o_ref[...] = jnp.maximum(o_ref[...], m8)


def _segmax(gate, batcht):
    return pl.pallas_call(
        _k_segmax,
        grid=(RB,),
        in_specs=[
            pl.BlockSpec((128, 128), lambda i: (i, 0)),
            pl.BlockSpec((128, 128), lambda i: (i, 0)),
        ],
        out_specs=pl.BlockSpec((8, B_PAD), lambda i: (0, 0)),
        out_shape=jax.ShapeDtypeStruct((8, B_PAD), jnp.float32),
    )(gate, batcht)


def _k_segsum(gate_ref, h_ref, batcht_ref, batchr_ref, smax_ref,
              num_ref, den_ref):
    i = pl.program_id(0)
    bt = batcht_ref[:, 0:1]
    lane = lax.broadcasted_iota(jnp.int32, (128, B_PAD), 1)
    oh = (bt == lane).astype(jnp.float32)
    br = batchr_ref[...].reshape(1, 128)
    sub = lax.broadcasted_iota(jnp.int32, (B_PAD, 128), 0)
    oht = (sub == br).astype(jnp.float32)
    smax_col = smax_ref[:, 0:1]
    gmax = jnp.dot(oh, smax_col, preferred_element_type=jnp.float32)
    e = jnp.exp(gate_ref[:, 0:1] - gmax)
    num = jnp.dot(oht, e * h_ref[...], preferred_element_type=jnp.float32)
    den = jnp.dot(oht, jnp.broadcast_to(e, (128, 128)),
                  preferred_element_type=jnp.float32)

    @pl.when(i == 0)
    def _():
        num_ref[...] = num
        den_ref[...] = den

    @pl.when(i > 0)
    def _():
        num_ref[...] += num
        den_ref[...] += den


def _segsum(gate, h, batcht, batchr3, smax8):
    smaxb = jnp.broadcast_to(smax8[0][:, None], (B_PAD, 128))
    return pl.pallas_call(
        _k_segsum,
        grid=(RB,),
        in_specs=[
            pl.BlockSpec((128, 128), lambda i: (i, 0)),
            pl.BlockSpec((128, 128), lambda i: (i, 0)),
            pl.BlockSpec((128, 128), lambda i: (i, 0)),
            pl.BlockSpec((1, 1, 128), lambda i: (i, 0, 0)),
            pl.BlockSpec((B_PAD, 128), lambda i: (0, 0)),
        ],
        out_specs=[
            pl.BlockSpec((B_PAD, 128), lambda i: (0, 0)),
            pl.BlockSpec((B_PAD, 128), lambda i: (0, 0)),
        ],
        out_shape=[
            jax.ShapeDtypeStruct((B_PAD, 128), jnp.float32),
            jax.ShapeDtypeStruct((B_PAD, 128), jnp.float32),
        ],
    )(gate, h, batcht, batchr3, smaxb)


def _k_proj(num_ref, den_ref, wp_ref, bp_ref, o_ref):
    den = den_ref[...]
    pooled = num_ref[...] / jnp.where(den > 0.0, den, 1.0)
    o_ref[...] = jnp.dot(
        pooled, wp_ref[...], preferred_element_type=jnp.float32) + bp_ref[0:1, :]


def _proj(num, den, wp, bp):
    bp8 = jnp.broadcast_to(bp.reshape(1, D), (8, D))
    return pl.pallas_call(
        _k_proj,
        grid=(8,),
        in_specs=[
            pl.BlockSpec((128, 128), lambda i: (i, 0)),
            pl.BlockSpec((128, 128), lambda i: (i, 0)),
            pl.BlockSpec((128, D), lambda i: (0, 0)),
            pl.BlockSpec((8, D), lambda i: (0, 0)),
        ],
        out_specs=pl.BlockSpec((128, D), lambda i: (i, 0)),
        out_shape=jax.ShapeDtypeStruct((B_PAD, D), jnp.float32),
    )(num, den, wp, bp8)


def _sc_pass(src_p, dst_p, zeros_sub, ones_k, g):
    chunks = [g[:, i * FC:(i + 1) * FC] for i in range(NCHUNK)]
    parts = _sc_spmm(src_p, dst_p, zeros_sub, ones_k, *chunks)
    y = parts[0] + parts[1]
    return y.transpose(1, 0, 2).reshape(N_PAD, H)


@jax.jit
def kernel(x, edge_index, batch, W0, b0, W1, b1, W2, b2, Wg, bg, Wp, bp):
    pad_e = E_PAD - E
    src_p = jnp.concatenate(
        [edge_index[0], jnp.full((pad_e,), N, jnp.int32)])
    dst_p = jnp.concatenate(
        [edge_index[1], jnp.full((pad_e,), N, jnp.int32)])
    xp = jnp.zeros((N_PAD, 128), jnp.float32).at[:N, :IN_F].set(x)
    w0p = jnp.zeros((128, H), jnp.float32).at[:IN_F].set(W0)
    zeros_sub = jnp.zeros((ROWS_SUB, FC), jnp.float32)
    ones_k = jnp.ones((K, FC), jnp.float32)
    dummy_g = jnp.zeros((N_PAD, FC), jnp.float32)

    deg_parts = _sc_deg(src_p, dst_p, zeros_sub, ones_k, dummy_g)
    deg = deg_parts[0, 0, :, 0] + deg_parts[1, 0, :, 0]
    degt = jnp.broadcast_to(deg[:, None], (N_PAD, 128))

    g = _pre(xp, w0p, degt)
    y = _sc_pass(src_p, dst_p, zeros_sub, ones_k, g)
    g = _mid(y, g, degt, b0, W1)
    y = _sc_pass(src_p, dst_p, zeros_sub, ones_k, g)
    g = _mid(y, g, degt, b1, W2)
    y = _sc_pass(src_p, dst_p, zeros_sub, ones_k, g)
    h3, gate = _post(y, g, degt, b2, Wg)

    batch_p = jnp.concatenate(
        [batch, jnp.full((N_PAD - N,), B, jnp.int32)])
    batcht = jnp.broadcast_to(batch_p[:, None], (N_PAD, 128))
    batchr3 = batch_p.reshape(RB, 1, 128)
    smax = _segmax(gate, batcht)
    num, den = _segsum(gate, h3, batcht, batchr3, smax)
    out = _proj(num, den, Wp, bp)
    return out[:B]
